# Initial kernel scaffold; baseline (speedup 1.0000x reference)
import jax
import jax.numpy as jnp
from jax.experimental import pallas as pl


def kernel(x, edge_index, W1, a1_src, a1_dst, b1, W2, a2_src, a2_dst, b2):
    raise NotImplementedError("write your pallas kernel here")


# speedup vs baseline: 11.1992x; 11.1992x over previous
import functools

import jax
import jax.numpy as jnp
from jax import lax
from jax.experimental import pallas as pl
from jax.experimental.pallas import tpu as pltpu
from jax.experimental.pallas import tpu_sc as plsc

N = 10000
D = 128
E = 320000

NC = 2
NS = 16
NP = 10240
ET = 10752
EP = NC * NS * ET
NBLK = 2 * ET // 128 // 8
NBLK_C0 = 11
ROWS_PER_TILE = NP // NS


def _mm_body(x_ref, w_ref, av_ref, h_ref, als_ref):
    h = jnp.dot(x_ref[...], w_ref[...], preferred_element_type=jnp.float32)
    h_ref[...] = h
    als_ref[...] = jnp.dot(h, av_ref[...], preferred_element_type=jnp.float32)


def _tc_mm(x, w, av, blk=1000):
    n = x.shape[0]
    return pl.pallas_call(
        _mm_body,
        grid=(n // blk,),
        in_specs=[
            pl.BlockSpec((blk, D), lambda i: (i, 0)),
            pl.BlockSpec((D, D), lambda i: (0, 0)),
            pl.BlockSpec((D, 2), lambda i: (0, 0)),
        ],
        out_specs=[
            pl.BlockSpec((blk, D), lambda i: (i, 0)),
            pl.BlockSpec((blk, 2), lambda i: (i, 0)),
        ],
        out_shape=[
            jax.ShapeDtypeStruct((n, D), jnp.float32),
            jax.ShapeDtypeStruct((n, 2), jnp.float32),
        ],
    )(x, w, av)


def _mid_body(p0_ref, p1_ref, b_ref, w_ref, av_ref, h_ref, als_ref):
    v = p0_ref[...] + p1_ref[...] + b_ref[...]
    h1e = jnp.where(v > 0, v, jnp.exp(jnp.minimum(v, 0.0)) - 1.0)
    h = jnp.dot(h1e, w_ref[...], preferred_element_type=jnp.float32)
    h_ref[...] = h
    als_ref[...] = jnp.dot(h, av_ref[...], preferred_element_type=jnp.float32)


def _tc_mid(p0, p1, b, w, av, blk=1000):
    n = p0.shape[0]
    return pl.pallas_call(
        _mid_body,
        grid=(n // blk,),
        in_specs=[
            pl.BlockSpec((blk, D), lambda i: (i, 0)),
            pl.BlockSpec((blk, D), lambda i: (i, 0)),
            pl.BlockSpec((1, D), lambda i: (0, 0)),
            pl.BlockSpec((D, D), lambda i: (0, 0)),
            pl.BlockSpec((D, 2), lambda i: (0, 0)),
        ],
        out_specs=[
            pl.BlockSpec((blk, D), lambda i: (i, 0)),
            pl.BlockSpec((blk, 2), lambda i: (i, 0)),
        ],
        out_shape=[
            jax.ShapeDtypeStruct((n, D), jnp.float32),
            jax.ShapeDtypeStruct((n, 2), jnp.float32),
        ],
    )(p0, p1, b, w, av)


def _fin_body(p0_ref, p1_ref, b_ref, o_ref):
    o_ref[...] = p0_ref[...] + p1_ref[...] + b_ref[...]


def _tc_fin(p0, p1, b, blk=1000):
    n = p0.shape[0]
    return pl.pallas_call(
        _fin_body,
        grid=(n // blk,),
        in_specs=[
            pl.BlockSpec((blk, D), lambda i: (i, 0)),
            pl.BlockSpec((blk, D), lambda i: (i, 0)),
            pl.BlockSpec((1, D), lambda i: (0, 0)),
        ],
        out_specs=pl.BlockSpec((blk, D), lambda i: (i, 0)),
        out_shape=jax.ShapeDtypeStruct((n, D), jnp.float32),
    )(p0, p1, b)


def _sc_body(h_hbm, als_hbm, ald_hbm, src_hbm, dst_hbm, pout_hbm,
             als_v, ald_v, src_b, dst_b, exb, s_st, rows, sem, s_sp, out_sp):
    c = lax.axis_index("c")
    sid = lax.axis_index("s")

    pltpu.sync_copy(als_hbm, als_v)
    pltpu.sync_copy(ald_hbm, ald_v)

    z16 = jnp.zeros((16,), jnp.float32)

    def _zero_rows(r, carry):
        for u in range(8):
            rows[r, pl.ds(u * 16, 16)] = z16
        return carry

    lax.fori_loop(0, 128, _zero_rows, 0)
    for i in range(ROWS_PER_TILE // 128):
        pltpu.sync_copy(rows, out_sp.at[pl.ds(sid * ROWS_PER_TILE + i * 128, 128)])

    @pl.when(sid == 0)
    def _():
        def _zero_s(i, carry):
            s_st[pl.ds(i * 16, 16)] = z16
            return carry
        lax.fori_loop(0, ROWS_PER_TILE // 16, _zero_s, 0)
        for i in range(NP // ROWS_PER_TILE):
            pltpu.sync_copy(s_st, s_sp.at[pl.ds(i * ROWS_PER_TILE, ROWS_PER_TILE)])

    plsc.subcore_barrier()

    def _edge_logits(src_blk, dst_blk, j):
        for u in range(8):
            si = src_blk[j, pl.ds(u * 16, 16)]
            di = dst_blk[j, pl.ds(u * 16, 16)]
            e = (plsc.load_gather(als_v, [si])
                 + plsc.load_gather(ald_v, [di]))
            e = jnp.where(e >= 0, e, 0.2 * e)
            exb[pl.ds(u * 16, 16)] = jnp.exp(e)

    def _phase_a(bi, carry):
        goff = (sid * NBLK + bi) * 8
        pltpu.sync_copy(src_hbm.at[pl.ds(goff, 8)], src_b)
        pltpu.sync_copy(dst_hbm.at[pl.ds(goff, 8)], dst_b)
        for j in range(8):
            _edge_logits(src_b, dst_b, j)
            pltpu.sync_copy(exb, s_sp.at[dst_b.at[j]], add=True)
        return carry

    lax.fori_loop(0, NBLK, _phase_a, 0)
    plsc.subcore_barrier()

    nb = jnp.where(c == 0, NBLK_C0, NBLK - NBLK_C0)

    def _phase_b(bi, carry):
        goff = (sid * NBLK + c * NBLK_C0 + bi) * 8
        pltpu.sync_copy(src_hbm.at[pl.ds(goff, 8)], src_b)
        pltpu.sync_copy(dst_hbm.at[pl.ds(goff, 8)], dst_b)
        for j in range(8):
            _edge_logits(src_b, dst_b, j)
            pltpu.async_copy(h_hbm.at[src_b.at[j]], rows, sem).wait()

            def _scale_rows(g, carry2):
                a16 = exb[pl.ds(g * 16, 16)]
                for k in range(16):
                    r = g * 16 + k
                    av = jnp.full((16,), a16[k])
                    for u in range(8):
                        rows[r, pl.ds(u * 16, 16)] = (
                            rows[r, pl.ds(u * 16, 16)] * av)
                return carry2

            lax.fori_loop(0, 8, _scale_rows, 0)
            pltpu.sync_copy(rows, out_sp.at[dst_b.at[j]], add=True)
        return carry

    lax.fori_loop(0, nb, _phase_b, 0)
    plsc.subcore_barrier()

    pltpu.sync_copy(s_sp.at[pl.ds(sid * ROWS_PER_TILE, ROWS_PER_TILE)], s_st)
    for i in range(ROWS_PER_TILE // 128):
        off = sid * ROWS_PER_TILE + i * 128
        pltpu.sync_copy(out_sp.at[pl.ds(off, 128)], rows)

        def _scale_out(g, carry):
            s16 = s_st[pl.ds(i * 128 + g * 16, 16)]
            r16 = 1.0 / (s16 + 1e-16)
            for k in range(16):
                r = g * 16 + k
                av = jnp.full((16,), r16[k])
                for u in range(8):
                    rows[r, pl.ds(u * 16, 16)] = rows[r, pl.ds(u * 16, 16)] * av
            return carry

        lax.fori_loop(0, 8, _scale_out, 0)
        pltpu.sync_copy(rows, pout_hbm.at[c, pl.ds(off, 128)])


@functools.partial(jax.jit, static_argnames=())
def _sc_layer(h, als_pad, ald_pad, src2, dst2):
    mesh = plsc.VectorSubcoreMesh(core_axis_name="c", subcore_axis_name="s")
    kern = pl.kernel(
        _sc_body,
        out_type=jax.ShapeDtypeStruct((NC, NP, D), jnp.float32),
        mesh=mesh,
        compiler_params=pltpu.CompilerParams(needs_layout_passes=False),
        scratch_types=[
            pltpu.VMEM((NP,), jnp.float32),
            pltpu.VMEM((NP,), jnp.float32),
            pltpu.VMEM((8, 128), jnp.int32),
            pltpu.VMEM((8, 128), jnp.int32),
            pltpu.VMEM((128,), jnp.float32),
            pltpu.VMEM((ROWS_PER_TILE,), jnp.float32),
            pltpu.VMEM((128, D), jnp.float32),
            pltpu.SemaphoreType.DMA,
            pltpu.VMEM_SHARED((NP,), jnp.float32),
            pltpu.VMEM_SHARED((NP, D), jnp.float32),
        ],
    )
    return kern(h, als_pad, ald_pad, src2, dst2)


def kernel(x, edge_index, W1, a1_src, a1_dst, b1, W2, a2_src, a2_dst, b2):
    sl = jnp.arange(N, dtype=edge_index.dtype)
    npad = EP - E - N
    src = jnp.concatenate([edge_index[0], sl,
                           jnp.zeros((npad,), edge_index.dtype)])
    dst = jnp.concatenate([edge_index[1], sl,
                           jnp.full((npad,), N, edge_index.dtype)])
    src2 = src.reshape(EP // 128, 128)
    dst2 = dst.reshape(EP // 128, 128)

    av1 = jnp.stack([a1_src[0], a1_dst[0]], axis=-1)
    av2 = jnp.stack([a2_src[0], a2_dst[0]], axis=-1)

    h1, als1 = _tc_mm(x, W1, av1)
    al1_pad = jnp.pad(als1, ((0, NP - N), (0, 0)))
    p1 = _sc_layer(h1, al1_pad[:, 0], al1_pad[:, 1], src2, dst2)

    h2, als2 = _tc_mid(p1[0, :N], p1[1, :N], b1.reshape(1, D), W2, av2)
    al2_pad = jnp.pad(als2, ((0, NP - N), (0, 0)))
    p2 = _sc_layer(h2, al2_pad[:, 0], al2_pad[:, 1], src2, dst2)

    return _tc_fin(p2[0, :N], p2[1, :N], b2.reshape(1, D))


# speedup vs baseline: 13.1867x; 1.1775x over previous
import functools

import jax
import jax.numpy as jnp
from jax import lax
from jax.experimental import pallas as pl
from jax.experimental.pallas import tpu as pltpu
from jax.experimental.pallas import tpu_sc as plsc

N = 10000
D = 128
E = 320000

NC = 2
NS = 16
NP = 10240
ET = 10752
EP = NC * NS * ET
NBLK = 2 * ET // 128 // 8
NBLK_C0 = 11
ROWS_PER_TILE = NP // NS


def _mm_body(x_ref, w_ref, av_ref, h_ref, als_ref):
    h = jnp.dot(x_ref[...], w_ref[...], preferred_element_type=jnp.float32)
    h_ref[...] = h
    als_ref[...] = jnp.dot(h, av_ref[...], preferred_element_type=jnp.float32)


def _tc_mm(x, w, av, blk=1000):
    n = x.shape[0]
    return pl.pallas_call(
        _mm_body,
        grid=(n // blk,),
        in_specs=[
            pl.BlockSpec((blk, D), lambda i: (i, 0)),
            pl.BlockSpec((D, D), lambda i: (0, 0)),
            pl.BlockSpec((D, 2), lambda i: (0, 0)),
        ],
        out_specs=[
            pl.BlockSpec((blk, D), lambda i: (i, 0)),
            pl.BlockSpec((blk, 2), lambda i: (i, 0)),
        ],
        out_shape=[
            jax.ShapeDtypeStruct((n, D), jnp.float32),
            jax.ShapeDtypeStruct((n, 2), jnp.float32),
        ],
    )(x, w, av)


def _mid_body(p0_ref, p1_ref, s_ref, b_ref, w_ref, av_ref, h_ref, als_ref):
    s = s_ref[...]
    den = s[:, 0:1] + s[:, 1:2] + 1e-16
    v = (p0_ref[...] + p1_ref[...]) / den + b_ref[...]
    h1e = jnp.where(v > 0, v, jnp.exp(jnp.minimum(v, 0.0)) - 1.0)
    h = jnp.dot(h1e, w_ref[...], preferred_element_type=jnp.float32)
    h_ref[...] = h
    als_ref[...] = jnp.dot(h, av_ref[...], preferred_element_type=jnp.float32)


def _tc_mid(p0, p1, s2, b, w, av, blk=1000):
    n = p0.shape[0]
    return pl.pallas_call(
        _mid_body,
        grid=(n // blk,),
        in_specs=[
            pl.BlockSpec((blk, D), lambda i: (i, 0)),
            pl.BlockSpec((blk, D), lambda i: (i, 0)),
            pl.BlockSpec((blk, 2), lambda i: (i, 0)),
            pl.BlockSpec((1, D), lambda i: (0, 0)),
            pl.BlockSpec((D, D), lambda i: (0, 0)),
            pl.BlockSpec((D, 2), lambda i: (0, 0)),
        ],
        out_specs=[
            pl.BlockSpec((blk, D), lambda i: (i, 0)),
            pl.BlockSpec((blk, 2), lambda i: (i, 0)),
        ],
        out_shape=[
            jax.ShapeDtypeStruct((n, D), jnp.float32),
            jax.ShapeDtypeStruct((n, 2), jnp.float32),
        ],
    )(p0, p1, s2, b, w, av)


def _fin_body(p0_ref, p1_ref, s_ref, b_ref, o_ref):
    s = s_ref[...]
    den = s[:, 0:1] + s[:, 1:2] + 1e-16
    o_ref[...] = (p0_ref[...] + p1_ref[...]) / den + b_ref[...]


def _tc_fin(p0, p1, s2, b, blk=1000):
    n = p0.shape[0]
    return pl.pallas_call(
        _fin_body,
        grid=(n // blk,),
        in_specs=[
            pl.BlockSpec((blk, D), lambda i: (i, 0)),
            pl.BlockSpec((blk, D), lambda i: (i, 0)),
            pl.BlockSpec((blk, 2), lambda i: (i, 0)),
            pl.BlockSpec((1, D), lambda i: (0, 0)),
        ],
        out_specs=pl.BlockSpec((blk, D), lambda i: (i, 0)),
        out_shape=jax.ShapeDtypeStruct((n, D), jnp.float32),
    )(p0, p1, s2, b)


def _sc_body(h_hbm, als_hbm, ald_hbm, src_hbm, dst_hbm, pout_hbm, sden_hbm,
             als_v, ald_v, src_b, dst_b, exb, zs, bufa, bufb,
             sem_s, sem_a, sem_b, s_sp, out_sp):
    c = lax.axis_index("c")
    sid = lax.axis_index("s")

    pltpu.sync_copy(als_hbm, als_v)
    pltpu.sync_copy(ald_hbm, ald_v)

    z16 = jnp.zeros((16,), jnp.float32)

    def _zero_buf(r, carry):
        for u in range(8):
            bufa[r, pl.ds(u * 16, 16)] = z16
        return carry

    lax.fori_loop(0, 64, _zero_buf, 0)
    for i in range(ROWS_PER_TILE // 64):
        pltpu.sync_copy(bufa, out_sp.at[pl.ds(sid * ROWS_PER_TILE + i * 64, 64)])

    @pl.when(sid == 0)
    def _():
        def _zero_s(i, carry):
            zs[pl.ds(i * 16, 16)] = z16
            return carry
        lax.fori_loop(0, ROWS_PER_TILE // 16, _zero_s, 0)
        for i in range(NP // ROWS_PER_TILE):
            pltpu.sync_copy(zs, s_sp.at[pl.ds(i * ROWS_PER_TILE, ROWS_PER_TILE)])

    plsc.subcore_barrier()

    nb = jnp.where(c == 0, NBLK_C0, NBLK - NBLK_C0)

    def _block(bi, carry):
        goff = (sid * NBLK + c * NBLK_C0 + bi) * 8
        pltpu.sync_copy(src_hbm.at[pl.ds(goff, 8)], src_b)
        pltpu.sync_copy(dst_hbm.at[pl.ds(goff, 8)], dst_b)
        for j in range(8):
            for u in range(8):
                si = src_b[j, pl.ds(u * 16, 16)]
                di = dst_b[j, pl.ds(u * 16, 16)]
                e = (plsc.load_gather(als_v, [si])
                     + plsc.load_gather(ald_v, [di]))
                e = jnp.where(e >= 0, e, 0.2 * e)
                exb[pl.ds(u * 16, 16)] = jnp.exp(e)
            sadd = pltpu.async_copy(exb, s_sp.at[dst_b.at[j]], add=True,
                                    sem=sem_s)
            ga = pltpu.async_copy(
                h_hbm.at[src_b.at[j, pl.ds(0, 64)]], bufa, sem_a)
            gb = pltpu.async_copy(
                h_hbm.at[src_b.at[j, pl.ds(64, 64)]], bufb, sem_b)

            def _scale(buf, base):
                def _scale_g(g, carry2):
                    a16 = exb[pl.ds(base + g * 16, 16)]
                    for k in range(16):
                        r = g * 16 + k
                        av = jnp.full((16,), a16[k])
                        for u in range(8):
                            buf[r, pl.ds(u * 16, 16)] = (
                                buf[r, pl.ds(u * 16, 16)] * av)
                    return carry2
                lax.fori_loop(0, 4, _scale_g, 0)

            ga.wait()
            _scale(bufa, 0)
            wa = pltpu.async_copy(
                bufa, out_sp.at[dst_b.at[j, pl.ds(0, 64)]], add=True,
                sem=sem_a)
            gb.wait()
            _scale(bufb, 64)
            wb = pltpu.async_copy(
                bufb, out_sp.at[dst_b.at[j, pl.ds(64, 64)]], add=True,
                sem=sem_b)
            sadd.wait()
            wa.wait()
            wb.wait()
        return carry

    lax.fori_loop(0, nb, _block, 0)
    plsc.subcore_barrier()

    off = sid * ROWS_PER_TILE
    pltpu.sync_copy(out_sp.at[pl.ds(off, ROWS_PER_TILE)],
                    pout_hbm.at[c, pl.ds(off, ROWS_PER_TILE)])
    pltpu.sync_copy(s_sp.at[pl.ds(off, ROWS_PER_TILE)],
                    sden_hbm.at[c, pl.ds(off, ROWS_PER_TILE)])


@functools.partial(jax.jit, static_argnames=())
def _sc_layer(h, als_pad, ald_pad, src2, dst2):
    mesh = plsc.VectorSubcoreMesh(core_axis_name="c", subcore_axis_name="s")
    kern = pl.kernel(
        _sc_body,
        out_type=[
            jax.ShapeDtypeStruct((NC, NP, D), jnp.float32),
            jax.ShapeDtypeStruct((NC, NP), jnp.float32),
        ],
        mesh=mesh,
        compiler_params=pltpu.CompilerParams(needs_layout_passes=False),
        scratch_types=[
            pltpu.VMEM((NP,), jnp.float32),
            pltpu.VMEM((NP,), jnp.float32),
            pltpu.VMEM((8, 128), jnp.int32),
            pltpu.VMEM((8, 128), jnp.int32),
            pltpu.VMEM((128,), jnp.float32),
            pltpu.VMEM((ROWS_PER_TILE,), jnp.float32),
            pltpu.VMEM((64, D), jnp.float32),
            pltpu.VMEM((64, D), jnp.float32),
            pltpu.SemaphoreType.DMA,
            pltpu.SemaphoreType.DMA,
            pltpu.SemaphoreType.DMA,
            pltpu.VMEM_SHARED((NP,), jnp.float32),
            pltpu.VMEM_SHARED((NP, D), jnp.float32),
        ],
    )
    return kern(h, als_pad, ald_pad, src2, dst2)


def kernel(x, edge_index, W1, a1_src, a1_dst, b1, W2, a2_src, a2_dst, b2):
    sl = jnp.arange(N, dtype=edge_index.dtype)
    npad = EP - E - N
    src = jnp.concatenate([edge_index[0], sl,
                           jnp.zeros((npad,), edge_index.dtype)])
    dst = jnp.concatenate([edge_index[1], sl,
                           jnp.full((npad,), N, edge_index.dtype)])
    src2 = src.reshape(EP // 128, 128)
    dst2 = dst.reshape(EP // 128, 128)

    av1 = jnp.stack([a1_src[0], a1_dst[0]], axis=-1)
    av2 = jnp.stack([a2_src[0], a2_dst[0]], axis=-1)

    h1, als1 = _tc_mm(x, W1, av1)
    al1_pad = jnp.pad(als1, ((0, NP - N), (0, 0)))
    p1, s1 = _sc_layer(h1, al1_pad[:, 0], al1_pad[:, 1], src2, dst2)
    s1t = jnp.stack([s1[0, :N], s1[1, :N]], axis=-1)

    h2, als2 = _tc_mid(p1[0, :N], p1[1, :N], s1t, b1.reshape(1, D), W2, av2)
    al2_pad = jnp.pad(als2, ((0, NP - N), (0, 0)))
    p2, s2 = _sc_layer(h2, al2_pad[:, 0], al2_pad[:, 1], src2, dst2)
    s2t = jnp.stack([s2[0, :N], s2[1, :N]], axis=-1)

    return _tc_fin(p2[0, :N], p2[1, :N], s2t, b2.reshape(1, D))
